# baseline (speedup 1.0000x reference)
import functools

import jax
import jax.numpy as jnp
from jax import lax
from jax.experimental import pallas as pl
from jax.experimental.pallas import tpu as pltpu
from jax.experimental.pallas import tpu_sc as plsc

N = 10000
E = 320000
K = 4
D_IN = 128
H = 16

NTILE = 16
NP = N + 112
RPT = NP // NTILE
EPT = 20480

CH16 = 128
NCH16 = EPT // CH16
GB16 = 8

CH128 = 64
NCH128 = EPT // CH128
GB128 = 2

NSUB = 8
SUBR = RPT // NSUB

RB = NP // 8

_mesh = plsc.VectorSubcoreMesh(
    core_axis_name="c", subcore_axis_name="s", num_cores=1)
_sc_params = pltpu.CompilerParams(use_tc_tiling_on_sc=False)


def _fill(buf, n, w, val):
    def body(i, _):
        buf[i, :] = jnp.full((w,), val, jnp.float32)
        return 0
    lax.fori_loop(0, n, body, 0)


@functools.partial(
    pl.kernel,
    out_type=jax.ShapeDtypeStruct((NP, H), jnp.float32),
    mesh=_mesh,
    compiler_params=_sc_params,
    scratch_types=[
        pltpu.VMEM((NCH16, CH16), jnp.int32),
        pltpu.VMEM((CH16, H), jnp.float32),
        pltpu.VMEM((RPT, H), jnp.float32),
        pltpu.VMEM_SHARED((NP, H), jnp.float32),
    ],
)
def _deg(col_hbm, deg_out, col_v, ones_v, buf_v, acc_sh):
    tid = lax.axis_index("s")
    base = tid * RPT
    pltpu.sync_copy(col_hbm.at[tid], col_v)
    _fill(ones_v, CH16, H, 1.0)
    _fill(buf_v, RPT, H, 0.0)
    pltpu.sync_copy(buf_v, acc_sh.at[pl.ds(base, RPT)])
    plsc.subcore_barrier()

    def edge_body(j, _):
        pltpu.sync_copy(ones_v, acc_sh.at[col_v.at[j]], add=True)
        return 0
    lax.fori_loop(0, NCH16, edge_body, 0)
    plsc.subcore_barrier()
    pltpu.sync_copy(acc_sh.at[pl.ds(base, RPT)], buf_v)
    pltpu.sync_copy(buf_v, deg_out.at[pl.ds(base, RPT)])


@functools.partial(
    pl.kernel,
    out_type=[
        jax.ShapeDtypeStruct((K, NP, H), jnp.float32),
        jax.ShapeDtypeStruct((NP, H), jnp.float32),
    ],
    mesh=_mesh,
    compiler_params=_sc_params,
    scratch_types=[
        pltpu.VMEM((NCH16, CH16), jnp.int32),
        pltpu.VMEM((NCH16, CH16), jnp.int32),
        pltpu.VMEM((GB16, CH16, H), jnp.float32),
        pltpu.VMEM((RPT, H), jnp.float32),
        pltpu.VMEM((RPT, H), jnp.float32),
        pltpu.VMEM((RPT, H), jnp.float32),
        pltpu.VMEM((RPT, H), jnp.float32),
        pltpu.VMEM_SHARED((NP, H), jnp.float32),
        pltpu.SemaphoreType.DMA,
    ],
)
def _hop16(row_hbm, col_hbm, dis_hbm, u0_hbm, cur_out, u_scr,
           row_v, col_v, gbuf_v, acc_v, dis_v, z_v, zero_v, acc_sh, gsem):
    tid = lax.axis_index("s")
    base = tid * RPT
    pltpu.sync_copy(row_hbm.at[tid], row_v)
    pltpu.sync_copy(col_hbm.at[tid], col_v)
    pltpu.sync_copy(dis_hbm.at[pl.ds(base, RPT)], dis_v)
    _fill(zero_v, RPT, H, 0.0)

    for k in range(K):
        u_src = u0_hbm if k == 0 else u_scr
        pltpu.sync_copy(zero_v, acc_sh.at[pl.ds(base, RPT)])
        plsc.subcore_barrier()

        def edge_group(g, _):
            descs = []
            for b in range(GB16):
                descs.append(pltpu.async_copy(
                    u_src.at[row_v.at[g * GB16 + b]], gbuf_v.at[b], gsem))
            for dd in descs:
                dd.wait()
            for b in range(GB16):
                pltpu.sync_copy(
                    gbuf_v.at[b], acc_sh.at[col_v.at[g * GB16 + b]], add=True)
            return 0
        lax.fori_loop(0, NCH16 // GB16, edge_group, 0)
        plsc.subcore_barrier()

        pltpu.sync_copy(acc_sh.at[pl.ds(base, RPT)], acc_v)

        def node_body(i, _):
            cv = dis_v[i, :] * acc_v[i, :]
            z_v[i, :] = cv
            acc_v[i, :] = dis_v[i, :] * cv
            return 0
        lax.fori_loop(0, RPT, node_body, 0)
        pltpu.sync_copy(z_v, cur_out.at[k].at[pl.ds(base, RPT)])
        if k < K - 1:
            pltpu.sync_copy(acc_v, u_scr.at[pl.ds(base, RPT)])


@functools.partial(
    pl.kernel,
    out_type=[
        jax.ShapeDtypeStruct((K, NP, D_IN), jnp.float32),
        jax.ShapeDtypeStruct((NP, D_IN), jnp.float32),
    ],
    mesh=_mesh,
    compiler_params=_sc_params,
    scratch_types=[
        pltpu.VMEM((GB128, CH128), jnp.int32),
        pltpu.VMEM((GB128, CH128), jnp.int32),
        pltpu.VMEM((GB128, CH128, D_IN), jnp.float32),
        pltpu.VMEM((SUBR, D_IN), jnp.float32),
        pltpu.VMEM((SUBR, D_IN), jnp.float32),
        pltpu.VMEM((SUBR, D_IN), jnp.float32),
        pltpu.VMEM_SHARED((NP, D_IN), jnp.float32),
        pltpu.SemaphoreType.DMA,
    ],
)
def _hop128(row_hbm, col_hbm, dis_hbm, u0_hbm, cur_out, u_scr,
            row_g, col_g, gbuf_v, acc_v, dis_v, z_v, acc_sh, gsem):
    tid = lax.axis_index("s")
    base = tid * RPT

    for k in range(K):
        u_src = u0_hbm if k == 0 else u_scr

        _fill(z_v, SUBR, D_IN, 0.0)

        def zero_body(j, _):
            pltpu.sync_copy(z_v, acc_sh.at[pl.ds(base + j * SUBR, SUBR)])
            return 0
        lax.fori_loop(0, NSUB, zero_body, 0)
        plsc.subcore_barrier()

        def edge_group(g, _):
            pltpu.sync_copy(
                row_hbm.at[tid].at[pl.ds(g * GB128, GB128)], row_g)
            pltpu.sync_copy(
                col_hbm.at[tid].at[pl.ds(g * GB128, GB128)], col_g)
            descs = []
            for b in range(GB128):
                descs.append(pltpu.async_copy(
                    u_src.at[row_g.at[b]], gbuf_v.at[b], gsem))
            for dd in descs:
                dd.wait()
            for b in range(GB128):
                pltpu.sync_copy(
                    gbuf_v.at[b], acc_sh.at[col_g.at[b]],
                    add=True)
            return 0
        lax.fori_loop(0, NCH128 // GB128, edge_group, 0)
        plsc.subcore_barrier()

        def node_chunk(j, _):
            off = base + j * SUBR
            pltpu.sync_copy(acc_sh.at[pl.ds(off, SUBR)], acc_v)
            pltpu.sync_copy(dis_hbm.at[pl.ds(off, SUBR)], dis_v)

            def node_body(i, _):
                for c in range(D_IN // H):
                    s = pl.ds(c * H, H)
                    cv = dis_v[i, s] * acc_v[i, s]
                    z_v[i, s] = cv
                    acc_v[i, s] = dis_v[i, s] * cv
                return 0
            lax.fori_loop(0, SUBR, node_body, 0)
            pltpu.sync_copy(z_v, cur_out.at[k].at[pl.ds(off, SUBR)])
            if k < K - 1:
                pltpu.sync_copy(acc_v, u_scr.at[pl.ds(off, SUBR)])
            return 0
        lax.fori_loop(0, NSUB, node_chunk, 0)


def _prep0_body(x_ref, w_ref, degr_ref, dis_ref, dis128_ref, u0_ref, y_ref):
    deg = degr_ref[...]
    ridx = (pl.program_id(0) * RB
            + lax.broadcasted_iota(jnp.int32, (RB, H), 0))
    safe = jnp.where(deg > 0, deg, 1.0)
    dis = jnp.where((deg > 0) & (ridx < N), 1.0 / jnp.sqrt(safe), 0.0)
    dis_ref[...] = dis
    dis128 = jnp.broadcast_to(dis[:, :1], (RB, D_IN))
    dis128_ref[...] = dis128
    xv = x_ref[...]
    u0_ref[...] = dis128 * xv
    y_ref[...] = jnp.dot(xv, w_ref[0], preferred_element_type=jnp.float32)


_prep0 = pl.pallas_call(
    _prep0_body,
    grid=(NP // RB,),
    in_specs=[
        pl.BlockSpec((RB, D_IN), lambda i: (i, 0)),
        pl.BlockSpec((K + 1, D_IN, H), lambda i: (0, 0, 0)),
        pl.BlockSpec((RB, H), lambda i: (i, 0)),
    ],
    out_specs=[
        pl.BlockSpec((RB, H), lambda i: (i, 0)),
        pl.BlockSpec((RB, D_IN), lambda i: (i, 0)),
        pl.BlockSpec((RB, D_IN), lambda i: (i, 0)),
        pl.BlockSpec((RB, H), lambda i: (i, 0)),
    ],
    out_shape=[
        jax.ShapeDtypeStruct((NP, H), jnp.float32),
        jax.ShapeDtypeStruct((NP, D_IN), jnp.float32),
        jax.ShapeDtypeStruct((NP, D_IN), jnp.float32),
        jax.ShapeDtypeStruct((NP, H), jnp.float32),
    ],
)


def _l1fin_body(y_ref, cur_ref, w_ref, b_ref, h_ref):
    out = y_ref[...]
    for k in range(1, K + 1):
        out = out + jnp.dot(cur_ref[k - 1], w_ref[k],
                            preferred_element_type=jnp.float32)
    out = out + b_ref[...]
    h_ref[...] = jnp.where(out >= 0, out, 0.01 * out)


_l1fin = pl.pallas_call(
    _l1fin_body,
    grid=(NP // RB,),
    in_specs=[
        pl.BlockSpec((RB, H), lambda i: (i, 0)),
        pl.BlockSpec((K, RB, D_IN), lambda i: (0, i, 0)),
        pl.BlockSpec((K + 1, D_IN, H), lambda i: (0, 0, 0)),
        pl.BlockSpec((1, H), lambda i: (0, 0)),
    ],
    out_specs=pl.BlockSpec((RB, H), lambda i: (i, 0)),
    out_shape=jax.ShapeDtypeStruct((NP, H), jnp.float32),
)


def _proj0_body(h_ref, w_ref, dis_ref, y_ref, u0_ref):
    hv = h_ref[...]
    y_ref[...] = jnp.dot(hv, w_ref[0], preferred_element_type=jnp.float32)
    u0_ref[...] = dis_ref[...] * hv


_proj0 = pl.pallas_call(
    _proj0_body,
    grid=(NP // RB,),
    in_specs=[
        pl.BlockSpec((RB, H), lambda i: (i, 0)),
        pl.BlockSpec((K + 1, H, H), lambda i: (0, 0, 0)),
        pl.BlockSpec((RB, H), lambda i: (i, 0)),
    ],
    out_specs=[
        pl.BlockSpec((RB, H), lambda i: (i, 0)),
        pl.BlockSpec((RB, H), lambda i: (i, 0)),
    ],
    out_shape=[
        jax.ShapeDtypeStruct((NP, H), jnp.float32),
        jax.ShapeDtypeStruct((NP, H), jnp.float32),
    ],
)


def _l2fin_body(y_ref, cur_ref, w_ref, b_ref, h_ref):
    out = y_ref[...]
    for k in range(1, K + 1):
        out = out + jnp.dot(cur_ref[k - 1], w_ref[k],
                            preferred_element_type=jnp.float32)
    out = out + b_ref[...]
    h_ref[...] = jnp.where(out >= 0, out, 0.01 * out)


_l2fin = pl.pallas_call(
    _l2fin_body,
    grid=(NP // RB,),
    in_specs=[
        pl.BlockSpec((RB, H), lambda i: (i, 0)),
        pl.BlockSpec((K, RB, H), lambda i: (0, i, 0)),
        pl.BlockSpec((K + 1, H, H), lambda i: (0, 0, 0)),
        pl.BlockSpec((1, H), lambda i: (0, 0)),
    ],
    out_specs=pl.BlockSpec((RB, H), lambda i: (i, 0)),
    out_shape=jax.ShapeDtypeStruct((NP, H), jnp.float32),
)


def _l3head_body(y_ref, cur_ref, w_ref, b_ref, wr_ref, br_ref, o_ref):
    out = y_ref[...]
    for k in range(1, K + 1):
        out = out + jnp.dot(cur_ref[k - 1], w_ref[k],
                            preferred_element_type=jnp.float32)
    out = out + b_ref[...]
    o_ref[...] = jnp.dot(out, wr_ref[...],
                         preferred_element_type=jnp.float32) + br_ref[...]


_l3head = pl.pallas_call(
    _l3head_body,
    grid=(NP // RB,),
    in_specs=[
        pl.BlockSpec((RB, H), lambda i: (i, 0)),
        pl.BlockSpec((K, RB, H), lambda i: (0, i, 0)),
        pl.BlockSpec((K + 1, H, H), lambda i: (0, 0, 0)),
        pl.BlockSpec((1, H), lambda i: (0, 0)),
        pl.BlockSpec((H, 1), lambda i: (0, 0)),
        pl.BlockSpec((1, 1), lambda i: (0, 0)),
    ],
    out_specs=pl.BlockSpec((RB, 1), lambda i: (i, 0)),
    out_shape=jax.ShapeDtypeStruct((NP, 1), jnp.float32),
)


def kernel(x, edge_index, edge_attr, W1, b1, W2, b2, W3, b3, Wr, br):
    del edge_attr
    row, col = edge_index[0], edge_index[1]
    ept_true = E // NTILE
    pad = EPT - ept_true
    pad_idx = (N + (jnp.arange(pad, dtype=jnp.int32) % (NP - N)))

    def part(idx, nch, ch):
        r = idx.reshape(NTILE, ept_true)
        p = jnp.broadcast_to(pad_idx[None, :], (NTILE, pad))
        return jnp.concatenate([r, p], axis=1).reshape(NTILE, nch, ch)

    row16 = part(row, NCH16, CH16)
    col16 = part(col, NCH16, CH16)
    row64 = part(row, NCH128, CH128)
    col64 = part(col, NCH128, CH128)
    x_pad = jnp.pad(x, ((0, NP - N), (0, 0)))

    deg_rows = _deg(col16)
    dis16, dis128, u0, y1 = _prep0(x_pad, W1, deg_rows)
    cur1, _ = _hop128(row64, col64, dis128, u0)
    h1 = _l1fin(y1, cur1, W1, b1.reshape(1, H))
    y2, u0a = _proj0(h1, W2, dis16)
    cur2, _ = _hop16(row16, col16, dis16, u0a)
    h2 = _l2fin(y2, cur2, W2, b2.reshape(1, H))
    y3, u0b = _proj0(h2, W3, dis16)
    cur3, _ = _hop16(row16, col16, dis16, u0b)
    out = _l3head(y3, cur3, W3, b3.reshape(1, H), Wr, br.reshape(1, 1))
    return out[:N]


# speedup vs baseline: 1.3513x; 1.3513x over previous
import functools

import jax
import jax.numpy as jnp
from jax import lax
from jax.experimental import pallas as pl
from jax.experimental.pallas import tpu as pltpu
from jax.experimental.pallas import tpu_sc as plsc

N = 10000
E = 320000
K = 4
D_IN = 128
H = 16

NTILE = 16
NP = N + 112
RPT = NP // NTILE
EPT = 20480

CH16 = 128
NCH16 = EPT // CH16
GB16 = 8

CH128 = 64
NCH128 = EPT // CH128
GB128 = 2

NSUB = 8
SUBR = RPT // NSUB

RB = NP // 8

_mesh = plsc.VectorSubcoreMesh(
    core_axis_name="c", subcore_axis_name="s", num_cores=1)
_sc_params = pltpu.CompilerParams(use_tc_tiling_on_sc=False)


def _fill(buf, n, w, val):
    def body(i, _):
        buf[i, :] = jnp.full((w,), val, jnp.float32)
        return 0
    lax.fori_loop(0, n, body, 0)


@functools.partial(
    pl.kernel,
    out_type=jax.ShapeDtypeStruct((NP, H), jnp.float32),
    mesh=_mesh,
    compiler_params=_sc_params,
    scratch_types=[
        pltpu.VMEM((NCH16, CH16), jnp.int32),
        pltpu.VMEM((CH16, H), jnp.float32),
        pltpu.VMEM((RPT, H), jnp.float32),
        pltpu.VMEM_SHARED((NP, H), jnp.float32),
    ],
)
def _deg(col_hbm, deg_out, col_v, ones_v, buf_v, acc_sh):
    tid = lax.axis_index("s")
    base = tid * RPT
    pltpu.sync_copy(col_hbm.at[tid], col_v)
    _fill(ones_v, CH16, H, 1.0)
    _fill(buf_v, RPT, H, 0.0)
    pltpu.sync_copy(buf_v, acc_sh.at[pl.ds(base, RPT)])
    plsc.subcore_barrier()

    def edge_body(j, _):
        pltpu.sync_copy(ones_v, acc_sh.at[col_v.at[j]], add=True)
        return 0
    lax.fori_loop(0, NCH16, edge_body, 0)
    plsc.subcore_barrier()
    pltpu.sync_copy(acc_sh.at[pl.ds(base, RPT)], buf_v)
    pltpu.sync_copy(buf_v, deg_out.at[pl.ds(base, RPT)])


@functools.partial(
    pl.kernel,
    out_type=[
        jax.ShapeDtypeStruct((K, NP, H), jnp.float32),
        jax.ShapeDtypeStruct((NP, H), jnp.float32),
    ],
    mesh=_mesh,
    compiler_params=_sc_params,
    scratch_types=[
        pltpu.VMEM((NCH16, CH16), jnp.int32),
        pltpu.VMEM((NCH16, CH16), jnp.int32),
        pltpu.VMEM((GB16, CH16, H), jnp.float32),
        pltpu.VMEM((RPT, H), jnp.float32),
        pltpu.VMEM((RPT, H), jnp.float32),
        pltpu.VMEM((RPT, H), jnp.float32),
        pltpu.VMEM((RPT, H), jnp.float32),
        pltpu.VMEM_SHARED((NP, H), jnp.float32),
        pltpu.SemaphoreType.DMA,
    ],
)
def _hop16(row_hbm, col_hbm, dis_hbm, u0_hbm, cur_out, u_scr,
           row_v, col_v, gbuf_v, acc_v, dis_v, z_v, zero_v, acc_sh, gsem):
    tid = lax.axis_index("s")
    base = tid * RPT
    pltpu.sync_copy(row_hbm.at[tid], row_v)
    pltpu.sync_copy(col_hbm.at[tid], col_v)
    pltpu.sync_copy(dis_hbm.at[pl.ds(base, RPT)], dis_v)
    _fill(zero_v, RPT, H, 0.0)

    for k in range(K):
        u_src = u0_hbm if k == 0 else u_scr
        pltpu.sync_copy(zero_v, acc_sh.at[pl.ds(base, RPT)])
        plsc.subcore_barrier()

        def edge_group(g, _):
            descs = []
            for b in range(GB16):
                descs.append(pltpu.async_copy(
                    u_src.at[row_v.at[g * GB16 + b]], gbuf_v.at[b], gsem))
            for dd in descs:
                dd.wait()
            for b in range(GB16):
                pltpu.sync_copy(
                    gbuf_v.at[b], acc_sh.at[col_v.at[g * GB16 + b]], add=True)
            return 0
        lax.fori_loop(0, NCH16 // GB16, edge_group, 0)
        plsc.subcore_barrier()

        pltpu.sync_copy(acc_sh.at[pl.ds(base, RPT)], acc_v)

        def node_body(i, _):
            cv = dis_v[i, :] * acc_v[i, :]
            z_v[i, :] = cv
            acc_v[i, :] = dis_v[i, :] * cv
            return 0
        lax.fori_loop(0, RPT, node_body, 0)
        pltpu.sync_copy(z_v, cur_out.at[k].at[pl.ds(base, RPT)])
        if k < K - 1:
            pltpu.sync_copy(acc_v, u_scr.at[pl.ds(base, RPT)])


@functools.partial(
    pl.kernel,
    out_type=[
        jax.ShapeDtypeStruct((K, NP, D_IN), jnp.float32),
        jax.ShapeDtypeStruct((NP, D_IN), jnp.float32),
    ],
    mesh=_mesh,
    compiler_params=_sc_params,
    scratch_types=[
        pltpu.VMEM((2, CH128), jnp.int32),
        pltpu.VMEM((2, CH128), jnp.int32),
        pltpu.VMEM((2, CH128, D_IN), jnp.float32),
        pltpu.VMEM((SUBR, D_IN), jnp.float32),
        pltpu.VMEM((SUBR, D_IN), jnp.float32),
        pltpu.VMEM((SUBR, D_IN), jnp.float32),
        pltpu.VMEM_SHARED((NP, D_IN), jnp.float32),
        pltpu.SemaphoreType.DMA,
        pltpu.SemaphoreType.DMA,
    ],
)
def _hop128(row_hbm, col_hbm, dis_hbm, u0_hbm, cur_out, u_scr,
            row_g, col_g, gbuf_v, acc_v, dis_v, z_v, acc_sh, gsem, isem):
    tid = lax.axis_index("s")
    base = tid * RPT

    def _wait_gather(u_src, p):
        pltpu.make_async_copy(
            u_src.at[row_g.at[p]], gbuf_v.at[p], gsem).wait()

    def _wait_idx(p):
        pltpu.make_async_copy(row_hbm.at[tid].at[0], row_g.at[p], isem).wait()
        pltpu.make_async_copy(col_hbm.at[tid].at[0], col_g.at[p], isem).wait()

    for k in range(K):
        u_src = u0_hbm if k == 0 else u_scr

        _fill(z_v, SUBR, D_IN, 0.0)

        def zero_body(j, _):
            pltpu.sync_copy(z_v, acc_sh.at[pl.ds(base + j * SUBR, SUBR)])
            return 0
        lax.fori_loop(0, NSUB, zero_body, 0)
        plsc.subcore_barrier()

        pltpu.sync_copy(row_hbm.at[tid].at[0], row_g.at[0])
        pltpu.sync_copy(col_hbm.at[tid].at[0], col_g.at[0])
        pltpu.async_copy(u_src.at[row_g.at[0]], gbuf_v.at[0], gsem)
        pltpu.async_copy(row_hbm.at[tid].at[1], row_g.at[1], isem)
        pltpu.async_copy(col_hbm.at[tid].at[1], col_g.at[1], isem)

        def edge_step(g, _):
            p = lax.rem(g, 2)
            q = lax.rem(g + 1, 2)
            _wait_idx(p)
            pltpu.async_copy(u_src.at[row_g.at[p]], gbuf_v.at[p], gsem)
            _wait_gather(u_src, q)
            pltpu.sync_copy(gbuf_v.at[q], acc_sh.at[col_g.at[q]], add=True)

            @pl.when(g + 1 < NCH128)
            def _():
                pltpu.async_copy(row_hbm.at[tid].at[g + 1], row_g.at[q], isem)
                pltpu.async_copy(col_hbm.at[tid].at[g + 1], col_g.at[q], isem)
            return 0
        lax.fori_loop(1, NCH128, edge_step, 0)
        last = (NCH128 - 1) % 2
        _wait_gather(u_src, last)
        pltpu.sync_copy(gbuf_v.at[last], acc_sh.at[col_g.at[last]], add=True)
        plsc.subcore_barrier()

        def node_chunk(j, _):
            off = base + j * SUBR
            pltpu.sync_copy(acc_sh.at[pl.ds(off, SUBR)], acc_v)
            pltpu.sync_copy(dis_hbm.at[pl.ds(off, SUBR)], dis_v)

            def node_body(i, _):
                for c in range(D_IN // H):
                    s = pl.ds(c * H, H)
                    cv = dis_v[i, s] * acc_v[i, s]
                    z_v[i, s] = cv
                    acc_v[i, s] = dis_v[i, s] * cv
                return 0
            lax.fori_loop(0, SUBR, node_body, 0)
            pltpu.sync_copy(z_v, cur_out.at[k].at[pl.ds(off, SUBR)])
            if k < K - 1:
                pltpu.sync_copy(acc_v, u_scr.at[pl.ds(off, SUBR)])
            return 0
        lax.fori_loop(0, NSUB, node_chunk, 0)


def _prep0_body(x_ref, w_ref, degr_ref, dis_ref, dis128_ref, u0_ref, y_ref):
    deg = degr_ref[...]
    ridx = (pl.program_id(0) * RB
            + lax.broadcasted_iota(jnp.int32, (RB, H), 0))
    safe = jnp.where(deg > 0, deg, 1.0)
    dis = jnp.where((deg > 0) & (ridx < N), 1.0 / jnp.sqrt(safe), 0.0)
    dis_ref[...] = dis
    dis128 = jnp.broadcast_to(dis[:, :1], (RB, D_IN))
    dis128_ref[...] = dis128
    xv = x_ref[...]
    u0_ref[...] = dis128 * xv
    y_ref[...] = jnp.dot(xv, w_ref[0], preferred_element_type=jnp.float32)


_prep0 = pl.pallas_call(
    _prep0_body,
    grid=(NP // RB,),
    in_specs=[
        pl.BlockSpec((RB, D_IN), lambda i: (i, 0)),
        pl.BlockSpec((K + 1, D_IN, H), lambda i: (0, 0, 0)),
        pl.BlockSpec((RB, H), lambda i: (i, 0)),
    ],
    out_specs=[
        pl.BlockSpec((RB, H), lambda i: (i, 0)),
        pl.BlockSpec((RB, D_IN), lambda i: (i, 0)),
        pl.BlockSpec((RB, D_IN), lambda i: (i, 0)),
        pl.BlockSpec((RB, H), lambda i: (i, 0)),
    ],
    out_shape=[
        jax.ShapeDtypeStruct((NP, H), jnp.float32),
        jax.ShapeDtypeStruct((NP, D_IN), jnp.float32),
        jax.ShapeDtypeStruct((NP, D_IN), jnp.float32),
        jax.ShapeDtypeStruct((NP, H), jnp.float32),
    ],
)


def _l1fin_body(y_ref, cur_ref, w_ref, b_ref, h_ref):
    out = y_ref[...]
    for k in range(1, K + 1):
        out = out + jnp.dot(cur_ref[k - 1], w_ref[k],
                            preferred_element_type=jnp.float32)
    out = out + b_ref[...]
    h_ref[...] = jnp.where(out >= 0, out, 0.01 * out)


_l1fin = pl.pallas_call(
    _l1fin_body,
    grid=(NP // RB,),
    in_specs=[
        pl.BlockSpec((RB, H), lambda i: (i, 0)),
        pl.BlockSpec((K, RB, D_IN), lambda i: (0, i, 0)),
        pl.BlockSpec((K + 1, D_IN, H), lambda i: (0, 0, 0)),
        pl.BlockSpec((1, H), lambda i: (0, 0)),
    ],
    out_specs=pl.BlockSpec((RB, H), lambda i: (i, 0)),
    out_shape=jax.ShapeDtypeStruct((NP, H), jnp.float32),
)


def _proj0_body(h_ref, w_ref, dis_ref, y_ref, u0_ref):
    hv = h_ref[...]
    y_ref[...] = jnp.dot(hv, w_ref[0], preferred_element_type=jnp.float32)
    u0_ref[...] = dis_ref[...] * hv


_proj0 = pl.pallas_call(
    _proj0_body,
    grid=(NP // RB,),
    in_specs=[
        pl.BlockSpec((RB, H), lambda i: (i, 0)),
        pl.BlockSpec((K + 1, H, H), lambda i: (0, 0, 0)),
        pl.BlockSpec((RB, H), lambda i: (i, 0)),
    ],
    out_specs=[
        pl.BlockSpec((RB, H), lambda i: (i, 0)),
        pl.BlockSpec((RB, H), lambda i: (i, 0)),
    ],
    out_shape=[
        jax.ShapeDtypeStruct((NP, H), jnp.float32),
        jax.ShapeDtypeStruct((NP, H), jnp.float32),
    ],
)


def _l2fin_body(y_ref, cur_ref, w_ref, b_ref, h_ref):
    out = y_ref[...]
    for k in range(1, K + 1):
        out = out + jnp.dot(cur_ref[k - 1], w_ref[k],
                            preferred_element_type=jnp.float32)
    out = out + b_ref[...]
    h_ref[...] = jnp.where(out >= 0, out, 0.01 * out)


_l2fin = pl.pallas_call(
    _l2fin_body,
    grid=(NP // RB,),
    in_specs=[
        pl.BlockSpec((RB, H), lambda i: (i, 0)),
        pl.BlockSpec((K, RB, H), lambda i: (0, i, 0)),
        pl.BlockSpec((K + 1, H, H), lambda i: (0, 0, 0)),
        pl.BlockSpec((1, H), lambda i: (0, 0)),
    ],
    out_specs=pl.BlockSpec((RB, H), lambda i: (i, 0)),
    out_shape=jax.ShapeDtypeStruct((NP, H), jnp.float32),
)


def _l3head_body(y_ref, cur_ref, w_ref, b_ref, wr_ref, br_ref, o_ref):
    out = y_ref[...]
    for k in range(1, K + 1):
        out = out + jnp.dot(cur_ref[k - 1], w_ref[k],
                            preferred_element_type=jnp.float32)
    out = out + b_ref[...]
    o_ref[...] = jnp.dot(out, wr_ref[...],
                         preferred_element_type=jnp.float32) + br_ref[...]


_l3head = pl.pallas_call(
    _l3head_body,
    grid=(NP // RB,),
    in_specs=[
        pl.BlockSpec((RB, H), lambda i: (i, 0)),
        pl.BlockSpec((K, RB, H), lambda i: (0, i, 0)),
        pl.BlockSpec((K + 1, H, H), lambda i: (0, 0, 0)),
        pl.BlockSpec((1, H), lambda i: (0, 0)),
        pl.BlockSpec((H, 1), lambda i: (0, 0)),
        pl.BlockSpec((1, 1), lambda i: (0, 0)),
    ],
    out_specs=pl.BlockSpec((RB, 1), lambda i: (i, 0)),
    out_shape=jax.ShapeDtypeStruct((NP, 1), jnp.float32),
)


def kernel(x, edge_index, edge_attr, W1, b1, W2, b2, W3, b3, Wr, br):
    del edge_attr
    row, col = edge_index[0], edge_index[1]
    ept_true = E // NTILE
    pad = EPT - ept_true
    pad_idx = (N + (jnp.arange(pad, dtype=jnp.int32) % (NP - N)))

    def part(idx, nch, ch):
        r = idx.reshape(NTILE, ept_true)
        p = jnp.broadcast_to(pad_idx[None, :], (NTILE, pad))
        return jnp.concatenate([r, p], axis=1).reshape(NTILE, nch, ch)

    row16 = part(row, NCH16, CH16)
    col16 = part(col, NCH16, CH16)
    row64 = part(row, NCH128, CH128)
    col64 = part(col, NCH128, CH128)
    x_pad = jnp.pad(x, ((0, NP - N), (0, 0)))

    deg_rows = _deg(col16)
    dis16, dis128, u0, y1 = _prep0(x_pad, W1, deg_rows)
    cur1, _ = _hop128(row64, col64, dis128, u0)
    h1 = _l1fin(y1, cur1, W1, b1.reshape(1, H))
    y2, u0a = _proj0(h1, W2, dis16)
    cur2, _ = _hop16(row16, col16, dis16, u0a)
    h2 = _l2fin(y2, cur2, W2, b2.reshape(1, H))
    y3, u0b = _proj0(h2, W3, dis16)
    cur3, _ = _hop16(row16, col16, dis16, u0b)
    out = _l3head(y3, cur3, W3, b3.reshape(1, H), Wr, br.reshape(1, 1))
    return out[:N]
